# baseline (speedup 1.0000x reference)
import functools

import jax
import jax.numpy as jnp
from jax import lax
from jax.experimental import pallas as pl
from jax.experimental.pallas import tpu as pltpu
from jax.experimental.pallas import tpu_sc as plsc

_N = 10000
_E = 160000
_DIM = 3
_LATENT = 128
_RADIUS = 0.025
_ACC_STD = 1.0
_ACC_MEAN = 0.0

_NBLK = 5000
_EBLK = 8000
_NC = 2
_NS = 16
_NW = _NC * _NS
_CH = 128

_f32 = jnp.float32


def _dot(a, b):
    return jnp.dot(a, b, preferred_element_type=_f32)


def _dot_bf(a, b):
    return jnp.dot(a.astype(jnp.bfloat16), b.astype(jnp.bfloat16),
                   preferred_element_type=_f32)


def _ln(y, g, b):
    mu = jnp.mean(y, axis=-1, keepdims=True)
    d = y - mu
    var = jnp.mean(d * d, axis=-1, keepdims=True)
    return d * lax.rsqrt(var + 1e-5) * g + b


def _sc_gather(table, idx, d):
    total = idx.shape[0]
    per_w = total // _NW
    nfull = per_w // _CH
    tail = per_w % _CH
    _SUB = 3
    sup = _SUB * _CH
    nsup = nfull // _SUB
    nleft = nfull % _SUB
    mesh = plsc.VectorSubcoreMesh(core_axis_name="c", subcore_axis_name="s")

    @functools.partial(
        pl.kernel,
        out_type=jax.ShapeDtypeStruct((total, d), _f32),
        mesh=mesh,
        scratch_types=[
            pltpu.VMEM((per_w,), jnp.int32),
            pltpu.VMEM((sup, d), _f32),
            pltpu.VMEM((sup, d), _f32),
            pltpu.SemaphoreType.DMA,
            pltpu.SemaphoreType.DMA,
            pltpu.SemaphoreType.DMA,
            pltpu.SemaphoreType.DMA,
        ],
    )
    def gk(table_hbm, idx_hbm, out_hbm, idx_v, r0, r1, g0, g1, w0, w1):
        w = lax.axis_index("s") * _NC + lax.axis_index("c")
        base = pl.multiple_of(w * per_w, 8)
        rows = (r0, r1)
        gsem = (g0, g1)
        wsem = (w0, w1)
        pltpu.sync_copy(idx_hbm.at[pl.ds(base, per_w)], idx_v)

        def fire_g(s, b):
            for j in range(_SUB):
                off = pl.multiple_of(s * sup + j * _CH, _CH)
                pltpu.async_copy(table_hbm.at[idx_v.at[pl.ds(off, _CH)]],
                                 rows[b].at[pl.ds(j * _CH, _CH)], gsem[b])

        def wait_g(b):
            pltpu.make_async_copy(table_hbm.at[pl.ds(0, sup)],
                                  rows[b], gsem[b]).wait()

        def fire_w(s, b):
            off = pl.multiple_of(base + s * sup, 8)
            pltpu.async_copy(rows[b], out_hbm.at[pl.ds(off, sup)], wsem[b])

        def wait_w(b):
            pltpu.make_async_copy(rows[b], out_hbm.at[pl.ds(base, sup)],
                                  wsem[b]).wait()

        def slot(s, b):
            wait_w(b)
            fire_g(s, b)
            wait_g(1 - b)
            fire_w(s - 1, 1 - b)

        fire_g(0, 0)
        fire_g(1, 1)
        wait_g(0)
        fire_w(0, 0)

        def pair(i, carry):
            slot(2 * i + 2, 0)
            slot(2 * i + 3, 1)
            return carry

        lax.fori_loop(0, (nsup - 2) // 2, pair, 0)
        if (nsup - 2) % 2:
            slot(nsup - 1, (nsup - 1) % 2)
        bl = (nsup - 1) % 2
        wait_g(bl)
        fire_w(nsup - 1, bl)
        wait_w(0)
        wait_w(1)
        for j in range(nleft):
            t = 3 * nsup + j
            off = pl.multiple_of(t * _CH, _CH)
            pltpu.async_copy(table_hbm.at[idx_v.at[pl.ds(off, _CH)]],
                             rows[0].at[pl.ds(0, _CH)], gsem[0]).wait()
            pltpu.async_copy(
                rows[0].at[pl.ds(0, _CH)],
                out_hbm.at[pl.ds(pl.multiple_of(base + t * _CH, 8), _CH)],
                wsem[0]).wait()
        if tail:
            toff = pl.multiple_of(nfull * _CH, 8)
            pltpu.async_copy(
                table_hbm.at[idx_v.at[pl.ds(toff, tail)]],
                rows[0].at[pl.ds(0, tail)], gsem[0]).wait()
            pltpu.async_copy(
                rows[0].at[pl.ds(0, tail)],
                out_hbm.at[pl.ds(pl.multiple_of(base + nfull * _CH, 8), tail)],
                wsem[0]).wait()

    return gk(table, idx)


_NPAD = 10240


def _sc_scatter_add(vals, idx, zeros_n):
    mesh = plsc.VectorSubcoreMesh(core_axis_name="c", subcore_axis_name="s")
    rows = _NPAD // _NS

    per_w = vals.shape[0] // _NW
    nfull = per_w // _CH
    tail = per_w % _CH
    rows_a = 624

    @functools.partial(
        pl.kernel,
        out_type=jax.ShapeDtypeStruct((_NC * _NPAD, _LATENT), _f32),
        mesh=mesh,
        scratch_types=[
            pltpu.VMEM((_CH,), jnp.int32),
            pltpu.VMEM((_CH,), jnp.int32),
            pltpu.VMEM((_CH,), jnp.int32),
            pltpu.VMEM((_CH, _LATENT), _f32),
            pltpu.VMEM((_CH, _LATENT), _f32),
            pltpu.VMEM((_CH, _LATENT), _f32),
            pltpu.VMEM((max(tail, 1),), jnp.int32),
            pltpu.VMEM_SHARED((_N, _LATENT), _f32),
            pltpu.SemaphoreType.DMA,
            pltpu.SemaphoreType.DMA,
            pltpu.SemaphoreType.DMA,
            pltpu.SemaphoreType.DMA,
            pltpu.SemaphoreType.DMA,
            pltpu.SemaphoreType.DMA,
            pltpu.SemaphoreType.DMA,
            pltpu.SemaphoreType.DMA,
            pltpu.SemaphoreType.DMA,
        ],
    )
    def sk(vals_hbm, idx_hbm, zeros_hbm, out_hbm, ib0, ib1, ib2, v0, v1, v2,
           it, acc_sh, is0, is1, is2, vs0, vs1, vs2, ss0, ss1, ss2):
        cid = lax.axis_index("c")
        sid = lax.axis_index("s")
        w = sid * _NC + cid
        base = pl.multiple_of(w * per_w, 8)
        idx_v = (ib0, ib1, ib2)
        vals_v = (v0, v1, v2)
        isem = (is0, is1, is2)
        vsem = (vs0, vs1, vs2)
        ssem = (ss0, ss1, ss2)
        zoff = pl.multiple_of(sid * rows_a, 8)

        def fire_l(t, b):
            off = pl.multiple_of(base + t * _CH, 8)
            pltpu.async_copy(idx_hbm.at[pl.ds(off, _CH)], idx_v[b], isem[b])
            pltpu.async_copy(vals_hbm.at[pl.ds(off, _CH)], vals_v[b], vsem[b])

        def wait_l(b):
            pltpu.make_async_copy(idx_hbm.at[pl.ds(base, _CH)],
                                  idx_v[b], isem[b]).wait()
            pltpu.make_async_copy(vals_hbm.at[pl.ds(base, _CH)],
                                  vals_v[b], vsem[b]).wait()

        def fire_scat(b):
            pltpu.async_copy(vals_v[b], acc_sh.at[idx_v[b]], ssem[b],
                             add=True)

        def wait_scat(b):
            pltpu.make_async_copy(vals_v[b], out_hbm.at[pl.ds(0, _CH)],
                                  ssem[b]).wait()

        def slot(t, b):
            wait_scat(b)
            fire_l(t, b)
            wait_l((b + 2) % 3)
            fire_scat((b + 2) % 3)

        fire_l(0, 0)
        fire_l(1, 1)
        fire_l(2, 2)
        pltpu.sync_copy(zeros_hbm.at[pl.ds(zoff, rows_a)],
                        acc_sh.at[pl.ds(zoff, rows_a)])

        @pl.when(sid == _NS - 1)
        def _():
            pltpu.sync_copy(zeros_hbm.at[pl.ds(16 * rows_a, _N - 16 * rows_a)],
                            acc_sh.at[pl.ds(16 * rows_a, _N - 16 * rows_a)])

        plsc.subcore_barrier()
        wait_l(0)
        fire_scat(0)
        wait_l(1)
        fire_scat(1)

        def trip(i, carry):
            slot(3 * i + 3, 0)
            slot(3 * i + 4, 1)
            slot(3 * i + 5, 2)
            return carry

        lax.fori_loop(0, (nfull - 3) // 3, trip, 0)
        for t in range(3 + 3 * ((nfull - 3) // 3), nfull):
            slot(t, t % 3)
        bl = (nfull - 1) % 3
        wait_l(bl)
        fire_scat(bl)
        wait_scat(0)
        wait_scat(1)
        wait_scat(2)
        if tail:
            toff = pl.multiple_of(base + nfull * _CH, 8)
            pltpu.async_copy(idx_hbm.at[pl.ds(toff, tail)], it, isem[0]).wait()
            pltpu.async_copy(vals_hbm.at[pl.ds(toff, tail)],
                             vals_v[0].at[pl.ds(0, tail)], vsem[0]).wait()
            pltpu.async_copy(vals_v[0].at[pl.ds(0, tail)],
                             acc_sh.at[it], vsem[0], add=True).wait()
        plsc.subcore_barrier()
        ooff = pl.multiple_of(cid * _NPAD + sid * rows_a, 8)
        pltpu.sync_copy(acc_sh.at[pl.ds(zoff, rows_a)],
                        out_hbm.at[pl.ds(ooff, rows_a)])

        @pl.when(sid == _NS - 1)
        def _():
            o2 = pl.multiple_of(cid * _NPAD + 16 * rows_a, 8)
            pltpu.sync_copy(
                acc_sh.at[pl.ds(16 * rows_a, _N - 16 * rows_a)],
                out_hbm.at[pl.ds(o2, _N - 16 * rows_a)])

    return sk(vals, idx, zeros_n)


def _row_spec(blk, d):
    return pl.BlockSpec((blk, d), lambda i: (i, 0))


def _row_spec_off(blk, d, off):
    return pl.BlockSpec((blk, d), lambda i: (i + off, 0))


def _w_spec(r, c):
    return pl.BlockSpec((r, c), lambda i: (0, 0))


def _node_enc_body(f_ref, w1, b1, w2, b2, w3, b3, g, bb, pa, pb,
                   x_ref, t_ref):
    h = jnp.maximum(_dot(f_ref[...], w1[...]) + b1[...], 0.0)
    h = jnp.maximum(_dot(h, w2[...]) + b2[...], 0.0)
    y = _ln(_dot(h, w3[...]) + b3[...], g[...], bb[...])
    x_ref[...] = y
    t_ref[:, :_LATENT] = _dot(y, pa[...])
    t_ref[:, _LATENT:] = _dot(y, pb[...])


def _node_enc_call(feats, w1, b1, w2, b2, w3, b3, g, bb, pa, pb):
    L = _LATENT
    return pl.pallas_call(
        _node_enc_body,
        grid=(_N // _NBLK,),
        in_specs=[_row_spec(_NBLK, 32), _w_spec(32, L), _w_spec(1, L),
                  _w_spec(L, L), _w_spec(1, L), _w_spec(L, L), _w_spec(1, L),
                  _w_spec(1, L), _w_spec(1, L), _w_spec(L, L), _w_spec(L, L)],
        out_specs=[_row_spec(_NBLK, L), _row_spec(_NBLK, 2 * L)],
        out_shape=[jax.ShapeDtypeStruct((_N, L), _f32),
                   jax.ShapeDtypeStruct((_N, 2 * L), _f32)],
    )(feats, w1, b1, w2, b2, w3, b3, g, bb, pa, pb)


def _edge_step1_body(ps_ref, pr_ref, gs_ref, gr_ref,
                     w1p, w1d, b1e, w2e, b2e, w3e, b3e, ge, bbe,
                     w1, b1, w2, b2, w3, b3, g, bb,
                     enew_ref, eupd_ref):
    d = (ps_ref[...] - pr_ref[...]) * (1.0 / _RADIUS)
    dist = jnp.sqrt(jnp.sum(d * d, axis=-1, keepdims=True) + 1e-12)
    h = _dot(d, w1p[...]) + dist * w1d[...] + b1e[...]
    h = jnp.maximum(h, 0.0)
    h = jnp.maximum(_dot(h, w2e[...]) + b2e[...], 0.0)
    e0 = _ln(_dot(h, w3e[...]) + b3e[...], ge[...], bbe[...])
    h = _dot_bf(e0, w1[...]) + gs_ref[...] + gr_ref[...] + b1[...]
    h = jnp.maximum(h, 0.0)
    h = jnp.maximum(_dot_bf(h, w2[...]) + b2[...], 0.0)
    y = _ln(_dot_bf(h, w3[...]) + b3[...], g[...], bb[...])
    eupd_ref[...] = y
    enew_ref[...] = e0 + y


def _edge_step1_call(pp, gg, encw, stepw):
    L = _LATENT
    off = _E // _EBLK
    return pl.pallas_call(
        _edge_step1_body,
        grid=(_E // _EBLK,),
        in_specs=[_row_spec(_EBLK, L), _row_spec_off(_EBLK, L, off),
                  _row_spec(_EBLK, L), _row_spec_off(_EBLK, L, off),
                  _w_spec(L, L), _w_spec(1, L), _w_spec(1, L),
                  _w_spec(L, L), _w_spec(1, L), _w_spec(L, L), _w_spec(1, L),
                  _w_spec(1, L), _w_spec(1, L),
                  _w_spec(L, L), _w_spec(1, L), _w_spec(L, L), _w_spec(1, L),
                  _w_spec(L, L), _w_spec(1, L), _w_spec(1, L), _w_spec(1, L)],
        out_specs=[_row_spec(_EBLK, L)] * 2,
        out_shape=[jax.ShapeDtypeStruct((_E, L), _f32)] * 2,
    )(pp, pp, gg, gg, *encw, *stepw)


def _edge_step_body(e_ref, gs_ref, gr_ref, w1, b1, w2, b2, w3, b3, g, bb,
                    enew_ref, eupd_ref):
    h = _dot_bf(e_ref[...], w1[...]) + gs_ref[...] + gr_ref[...] + b1[...]
    h = jnp.maximum(h, 0.0)
    h = jnp.maximum(_dot_bf(h, w2[...]) + b2[...], 0.0)
    y = _ln(_dot_bf(h, w3[...]) + b3[...], g[...], bb[...])
    eupd_ref[...] = y
    enew_ref[...] = e_ref[...] + y


def _edge_step_call(e, gg, w1, b1, w2, b2, w3, b3, g, bb):
    L = _LATENT
    ne = e.shape[0]
    off = ne // _EBLK
    return pl.pallas_call(
        _edge_step_body,
        grid=(ne // _EBLK,),
        in_specs=[_row_spec(_EBLK, L), _row_spec(_EBLK, L),
                  _row_spec_off(_EBLK, L, off)] +
                 [_w_spec(L, L), _w_spec(1, L), _w_spec(L, L), _w_spec(1, L),
                  _w_spec(L, L), _w_spec(1, L), _w_spec(1, L), _w_spec(1, L)],
        out_specs=[_row_spec(_EBLK, L)] * 2,
        out_shape=[jax.ShapeDtypeStruct((ne, L), _f32)] * 2,
    )(e, gg, gg, w1, b1, w2, b2, w3, b3, g, bb)


def _node_step_proj_body(x_ref, s0_ref, s1_ref, v1x, v1a, b1,
                         w2, b2, w3, b3, g, bb, pa, pb, xn_ref, t_ref):
    agg = s0_ref[...] + s1_ref[...]
    h = _dot(x_ref[...], v1x[...]) + _dot(agg, v1a[...]) + b1[...]
    h = jnp.maximum(h, 0.0)
    h = jnp.maximum(_dot(h, w2[...]) + b2[...], 0.0)
    y = _ln(_dot(h, w3[...]) + b3[...], g[...], bb[...])
    xn = x_ref[...] + y
    xn_ref[...] = xn
    t_ref[:, :_LATENT] = _dot(xn, pa[...])
    t_ref[:, _LATENT:] = _dot(xn, pb[...])


def _node_step_proj_call(x, ss, v1x, v1a, b1, w2, b2, w3, b3, g, bb, pa, pb):
    L = _LATENT
    return pl.pallas_call(
        _node_step_proj_body,
        grid=(_N // _NBLK,),
        in_specs=[_row_spec(_NBLK, L)] * 3 +
                 [_w_spec(L, L), _w_spec(L, L), _w_spec(1, L), _w_spec(L, L),
                  _w_spec(1, L), _w_spec(L, L), _w_spec(1, L), _w_spec(1, L),
                  _w_spec(1, L), _w_spec(L, L), _w_spec(L, L)],
        out_specs=[_row_spec(_NBLK, L), _row_spec(_NBLK, 2 * L)],
        out_shape=[jax.ShapeDtypeStruct((_N, L), _f32),
                   jax.ShapeDtypeStruct((_N, 2 * L), _f32)],
    )(x, *ss, v1x, v1a, b1, w2, b2, w3, b3, g, bb, pa, pb)


def _node_dec_body(x_ref, s0_ref, s1_ref, v1x, v1a, b1, w2, b2, w3, b3,
                   g, bb, d1, db1, d2, db2, d3, db3, out_ref):
    agg = s0_ref[...] + s1_ref[...]
    h = _dot(x_ref[...], v1x[...]) + _dot(agg, v1a[...]) + b1[...]
    h = jnp.maximum(h, 0.0)
    h = jnp.maximum(_dot(h, w2[...]) + b2[...], 0.0)
    y = _ln(_dot(h, w3[...]) + b3[...], g[...], bb[...])
    xn = x_ref[...] + y
    h = jnp.maximum(_dot(xn, d1[...]) + db1[...], 0.0)
    h = jnp.maximum(_dot(h, d2[...]) + db2[...], 0.0)
    out_ref[...] = _dot(h, d3[...]) + db3[...]


def _node_dec_call(x, ss, v1x, v1a, b1, w2, b2, w3, b3, g, bb,
                   d1, db1, d2, db2, d3, db3):
    L = _LATENT
    return pl.pallas_call(
        _node_dec_body,
        grid=(_N // _NBLK,),
        in_specs=[_row_spec(_NBLK, L)] * 3 +
                 [_w_spec(L, L), _w_spec(L, L), _w_spec(1, L), _w_spec(L, L),
                  _w_spec(1, L), _w_spec(L, L), _w_spec(1, L), _w_spec(1, L),
                  _w_spec(1, L), _w_spec(L, L), _w_spec(1, L), _w_spec(L, L),
                  _w_spec(1, L), _w_spec(L, L), _w_spec(1, L)],
        out_specs=_row_spec(_NBLK, L),
        out_shape=jax.ShapeDtypeStruct((_N, L), _f32),
    )(x, *ss, v1x, v1a, b1, w2, b2, w3, b3, g, bb, d1, db1, d2, db2, d3, db3)


def _r1(v):
    return v.reshape(1, -1)


def kernel(current_positions, edge_index, particle_types, meta_feature, params):
    n = current_positions.shape[0]
    assert n == _N and edge_index.shape[1] == _E

    senders = edge_index[0]
    receivers = edge_index[1]
    idx_pos = jnp.concatenate([senders, receivers])
    idx_lat = jnp.concatenate([2 * senders, 2 * receivers + 1])

    most_recent = current_positions[:, -1]
    flat_vel = (current_positions[:, 1:] - current_positions[:, :-1]).reshape(n, 15)
    npos = (most_recent - 0.5) / 0.5

    nenc = params["node_enc"]["mlp"]
    w1n = nenc[0]["W"]
    meta4 = meta_feature[jnp.array([0, 1, 2, 5])]
    b1n = nenc[0]["b"] + meta4 @ w1n[18:22]
    w1n_eff = jnp.concatenate(
        [w1n[0:18], w1n[22:23], params["embed"] @ w1n[23:39],
         jnp.zeros((4, _LATENT), _f32)], axis=0)
    onehot = (particle_types[:, None] == jnp.arange(9)[None, :]).astype(_f32)
    mcol = jnp.where(particle_types == 1, 10.0, meta_feature[6])
    feats = jnp.concatenate(
        [flat_vel, npos, mcol[:, None], onehot, jnp.zeros((n, 4), _f32)], axis=1)

    gnn = params["gnn"]
    p0 = gnn[0]["edge_mlp"][0]["W"]
    x, xt = _node_enc_call(
        feats, w1n_eff, _r1(b1n), nenc[1]["W"], _r1(nenc[1]["b"]),
        nenc[2]["W"], _r1(nenc[2]["b"]),
        _r1(params["node_enc"]["ln"]["g"]), _r1(params["node_enc"]["ln"]["b"]),
        p0[_LATENT:2 * _LATENT], p0[2 * _LATENT:])

    postable = jnp.concatenate(
        [most_recent, jnp.zeros((n, _LATENT - _DIM), _f32)], axis=1)
    eenc = params["edge_enc"]["mlp"]
    w1e4 = eenc[0]["W"]
    w1p = jnp.concatenate(
        [w1e4[0:3], jnp.zeros((_LATENT - 3, _LATENT), _f32)], axis=0)
    encw = (w1p, w1e4[3:4], _r1(eenc[0]["b"]),
            eenc[1]["W"], _r1(eenc[1]["b"]), eenc[2]["W"], _r1(eenc[2]["b"]),
            _r1(params["edge_enc"]["ln"]["g"]),
            _r1(params["edge_enc"]["ln"]["b"]))
    dec = params["decode"]["mlp"]
    w3d = jnp.zeros((_LATENT, _LATENT), _f32).at[:, :_DIM + 1].set(dec[2]["W"])
    b3d = jnp.zeros((_LATENT,), _f32).at[:_DIM + 1].set(dec[2]["b"])

    zeros_n = jnp.zeros((_NPAD, _LATENT), _f32)
    e = None
    pred = None
    for k in range(len(gnn)):
        blk = gnn[k]
        em = blk["edge_mlp"]
        nm = blk["node_mlp"]
        table = xt.reshape(2 * n, _LATENT)
        estep = (em[0]["W"][:_LATENT], _r1(em[0]["b"]),
                 em[1]["W"], _r1(em[1]["b"]), em[2]["W"], _r1(em[2]["b"]),
                 _r1(blk["edge_ln"]["g"]), _r1(blk["edge_ln"]["b"]))
        if k == 0:
            pp = _sc_gather(postable, idx_pos, _LATENT)
            gg = _sc_gather(table, idx_lat, _LATENT)
            e, e_upd = _edge_step1_call(pp, gg, encw, estep)
        else:
            gg = _sc_gather(table, idx_lat, _LATENT)
            e, e_upd = _edge_step_call(e, gg, *estep)
        s_part = _sc_scatter_add(e_upd, receivers, zeros_n)
        ss = [s_part[:n], s_part[_NPAD:_NPAD + n]]
        nargs = (x, ss, nm[0]["W"][:_LATENT], nm[0]["W"][_LATENT:],
                 _r1(nm[0]["b"]), nm[1]["W"], _r1(nm[1]["b"]),
                 nm[2]["W"], _r1(nm[2]["b"]),
                 _r1(blk["node_ln"]["g"]), _r1(blk["node_ln"]["b"]))
        if k + 1 < len(gnn):
            pn = gnn[k + 1]["edge_mlp"][0]["W"]
            x, xt = _node_step_proj_call(
                *nargs, pn[_LATENT:2 * _LATENT], pn[2 * _LATENT:])
        else:
            pred = _node_dec_call(
                *nargs, dec[0]["W"], _r1(dec[0]["b"]),
                dec[1]["W"], _r1(dec[1]["b"]), w3d, _r1(b3d))
    acc = pred[:, :_DIM] * _ACC_STD + _ACC_MEAN
    recent_vel = most_recent - current_positions[:, -2]
    return most_recent + recent_vel + acc


# speedup vs baseline: 1.0022x; 1.0022x over previous
import functools

import jax
import jax.numpy as jnp
from jax import lax
from jax.experimental import pallas as pl
from jax.experimental.pallas import tpu as pltpu
from jax.experimental.pallas import tpu_sc as plsc

_N = 10000
_E = 160000
_DIM = 3
_LATENT = 128
_RADIUS = 0.025
_ACC_STD = 1.0
_ACC_MEAN = 0.0

_NBLK = 5000
_EBLK = 8000
_NC = 2
_NS = 16
_NW = _NC * _NS
_CH = 128

_f32 = jnp.float32


def _dot(a, b):
    return jnp.dot(a, b, preferred_element_type=_f32)


def _ln(y, g, b):
    mu = jnp.mean(y, axis=-1, keepdims=True)
    d = y - mu
    var = jnp.mean(d * d, axis=-1, keepdims=True)
    return d * lax.rsqrt(var + 1e-5) * g + b


def _sc_gather(table, idx, d):
    total = idx.shape[0]
    per_w = total // _NW
    nfull = per_w // _CH
    tail = per_w % _CH
    _SUB = 3
    sup = _SUB * _CH
    nsup = nfull // _SUB
    nleft = nfull % _SUB
    mesh = plsc.VectorSubcoreMesh(core_axis_name="c", subcore_axis_name="s")

    @functools.partial(
        pl.kernel,
        out_type=jax.ShapeDtypeStruct((total, d), _f32),
        mesh=mesh,
        scratch_types=[
            pltpu.VMEM((per_w,), jnp.int32),
            pltpu.VMEM((sup, d), _f32),
            pltpu.VMEM((sup, d), _f32),
            pltpu.SemaphoreType.DMA,
            pltpu.SemaphoreType.DMA,
            pltpu.SemaphoreType.DMA,
            pltpu.SemaphoreType.DMA,
        ],
    )
    def gk(table_hbm, idx_hbm, out_hbm, idx_v, r0, r1, g0, g1, w0, w1):
        w = lax.axis_index("s") * _NC + lax.axis_index("c")
        base = pl.multiple_of(w * per_w, 8)
        rows = (r0, r1)
        gsem = (g0, g1)
        wsem = (w0, w1)
        pltpu.sync_copy(idx_hbm.at[pl.ds(base, per_w)], idx_v)

        def fire_g(s, b):
            for j in range(_SUB):
                off = pl.multiple_of(s * sup + j * _CH, _CH)
                pltpu.async_copy(table_hbm.at[idx_v.at[pl.ds(off, _CH)]],
                                 rows[b].at[pl.ds(j * _CH, _CH)], gsem[b])

        def wait_g(b):
            pltpu.make_async_copy(table_hbm.at[pl.ds(0, sup)],
                                  rows[b], gsem[b]).wait()

        def fire_w(s, b):
            off = pl.multiple_of(base + s * sup, 8)
            pltpu.async_copy(rows[b], out_hbm.at[pl.ds(off, sup)], wsem[b])

        def wait_w(b):
            pltpu.make_async_copy(rows[b], out_hbm.at[pl.ds(base, sup)],
                                  wsem[b]).wait()

        def slot(s, b):
            wait_w(b)
            fire_g(s, b)
            wait_g(1 - b)
            fire_w(s - 1, 1 - b)

        fire_g(0, 0)
        fire_g(1, 1)
        wait_g(0)
        fire_w(0, 0)

        def pair(i, carry):
            slot(2 * i + 2, 0)
            slot(2 * i + 3, 1)
            return carry

        lax.fori_loop(0, (nsup - 2) // 2, pair, 0)
        if (nsup - 2) % 2:
            slot(nsup - 1, (nsup - 1) % 2)
        bl = (nsup - 1) % 2
        wait_g(bl)
        fire_w(nsup - 1, bl)
        wait_w(0)
        wait_w(1)
        for j in range(nleft):
            t = 3 * nsup + j
            off = pl.multiple_of(t * _CH, _CH)
            pltpu.async_copy(table_hbm.at[idx_v.at[pl.ds(off, _CH)]],
                             rows[0].at[pl.ds(0, _CH)], gsem[0]).wait()
            pltpu.async_copy(
                rows[0].at[pl.ds(0, _CH)],
                out_hbm.at[pl.ds(pl.multiple_of(base + t * _CH, 8), _CH)],
                wsem[0]).wait()
        if tail:
            toff = pl.multiple_of(nfull * _CH, 8)
            pltpu.async_copy(
                table_hbm.at[idx_v.at[pl.ds(toff, tail)]],
                rows[0].at[pl.ds(0, tail)], gsem[0]).wait()
            pltpu.async_copy(
                rows[0].at[pl.ds(0, tail)],
                out_hbm.at[pl.ds(pl.multiple_of(base + nfull * _CH, 8), tail)],
                wsem[0]).wait()

    return gk(table, idx)


_NPAD = 10240


def _sc_scatter_add(vals, idx, zeros_n):
    mesh = plsc.VectorSubcoreMesh(core_axis_name="c", subcore_axis_name="s")
    rows = _NPAD // _NS

    per_w = vals.shape[0] // _NW
    nfull = per_w // _CH
    tail = per_w % _CH
    rows_a = 624

    @functools.partial(
        pl.kernel,
        out_type=jax.ShapeDtypeStruct((_NC * _NPAD, _LATENT), _f32),
        mesh=mesh,
        scratch_types=[
            pltpu.VMEM((_CH,), jnp.int32),
            pltpu.VMEM((_CH,), jnp.int32),
            pltpu.VMEM((_CH,), jnp.int32),
            pltpu.VMEM((_CH, _LATENT), _f32),
            pltpu.VMEM((_CH, _LATENT), _f32),
            pltpu.VMEM((_CH, _LATENT), _f32),
            pltpu.VMEM((max(tail, 1),), jnp.int32),
            pltpu.VMEM_SHARED((_N, _LATENT), _f32),
            pltpu.SemaphoreType.DMA,
            pltpu.SemaphoreType.DMA,
            pltpu.SemaphoreType.DMA,
            pltpu.SemaphoreType.DMA,
            pltpu.SemaphoreType.DMA,
            pltpu.SemaphoreType.DMA,
            pltpu.SemaphoreType.DMA,
            pltpu.SemaphoreType.DMA,
            pltpu.SemaphoreType.DMA,
        ],
    )
    def sk(vals_hbm, idx_hbm, zeros_hbm, out_hbm, ib0, ib1, ib2, v0, v1, v2,
           it, acc_sh, is0, is1, is2, vs0, vs1, vs2, ss0, ss1, ss2):
        cid = lax.axis_index("c")
        sid = lax.axis_index("s")
        w = sid * _NC + cid
        base = pl.multiple_of(w * per_w, 8)
        idx_v = (ib0, ib1, ib2)
        vals_v = (v0, v1, v2)
        isem = (is0, is1, is2)
        vsem = (vs0, vs1, vs2)
        ssem = (ss0, ss1, ss2)
        zoff = pl.multiple_of(sid * rows_a, 8)

        def fire_l(t, b):
            off = pl.multiple_of(base + t * _CH, 8)
            pltpu.async_copy(idx_hbm.at[pl.ds(off, _CH)], idx_v[b], isem[b])
            pltpu.async_copy(vals_hbm.at[pl.ds(off, _CH)], vals_v[b], vsem[b])

        def wait_l(b):
            pltpu.make_async_copy(idx_hbm.at[pl.ds(base, _CH)],
                                  idx_v[b], isem[b]).wait()
            pltpu.make_async_copy(vals_hbm.at[pl.ds(base, _CH)],
                                  vals_v[b], vsem[b]).wait()

        def fire_scat(b):
            pltpu.async_copy(vals_v[b], acc_sh.at[idx_v[b]], ssem[b],
                             add=True)

        def wait_scat(b):
            pltpu.make_async_copy(vals_v[b], out_hbm.at[pl.ds(0, _CH)],
                                  ssem[b]).wait()

        def slot(t, b):
            wait_scat(b)
            fire_l(t, b)
            wait_l((b + 2) % 3)
            fire_scat((b + 2) % 3)

        fire_l(0, 0)
        fire_l(1, 1)
        fire_l(2, 2)
        pltpu.sync_copy(zeros_hbm.at[pl.ds(zoff, rows_a)],
                        acc_sh.at[pl.ds(zoff, rows_a)])

        @pl.when(sid == _NS - 1)
        def _():
            pltpu.sync_copy(zeros_hbm.at[pl.ds(16 * rows_a, _N - 16 * rows_a)],
                            acc_sh.at[pl.ds(16 * rows_a, _N - 16 * rows_a)])

        plsc.subcore_barrier()
        wait_l(0)
        fire_scat(0)
        wait_l(1)
        fire_scat(1)

        def trip(i, carry):
            slot(3 * i + 3, 0)
            slot(3 * i + 4, 1)
            slot(3 * i + 5, 2)
            return carry

        lax.fori_loop(0, (nfull - 3) // 3, trip, 0)
        for t in range(3 + 3 * ((nfull - 3) // 3), nfull):
            slot(t, t % 3)
        bl = (nfull - 1) % 3
        wait_l(bl)
        fire_scat(bl)
        wait_scat(0)
        wait_scat(1)
        wait_scat(2)
        if tail:
            toff = pl.multiple_of(base + nfull * _CH, 8)
            pltpu.async_copy(idx_hbm.at[pl.ds(toff, tail)], it, isem[0]).wait()
            pltpu.async_copy(vals_hbm.at[pl.ds(toff, tail)],
                             vals_v[0].at[pl.ds(0, tail)], vsem[0]).wait()
            pltpu.async_copy(vals_v[0].at[pl.ds(0, tail)],
                             acc_sh.at[it], vsem[0], add=True).wait()
        plsc.subcore_barrier()
        ooff = pl.multiple_of(cid * _NPAD + sid * rows_a, 8)
        pltpu.sync_copy(acc_sh.at[pl.ds(zoff, rows_a)],
                        out_hbm.at[pl.ds(ooff, rows_a)])

        @pl.when(sid == _NS - 1)
        def _():
            o2 = pl.multiple_of(cid * _NPAD + 16 * rows_a, 8)
            pltpu.sync_copy(
                acc_sh.at[pl.ds(16 * rows_a, _N - 16 * rows_a)],
                out_hbm.at[pl.ds(o2, _N - 16 * rows_a)])

    return sk(vals, idx, zeros_n)


def _row_spec(blk, d):
    return pl.BlockSpec((blk, d), lambda i: (i, 0))


def _row_spec_off(blk, d, off):
    return pl.BlockSpec((blk, d), lambda i: (i + off, 0))


def _w_spec(r, c):
    return pl.BlockSpec((r, c), lambda i: (0, 0))


def _node_enc_body(f_ref, w1, b1, w2, b2, w3, b3, g, bb, pa, pb,
                   x_ref, t_ref):
    h = jnp.maximum(_dot(f_ref[...], w1[...]) + b1[...], 0.0)
    h = jnp.maximum(_dot(h, w2[...]) + b2[...], 0.0)
    y = _ln(_dot(h, w3[...]) + b3[...], g[...], bb[...])
    x_ref[...] = y
    t_ref[:, :_LATENT] = _dot(y, pa[...])
    t_ref[:, _LATENT:] = _dot(y, pb[...])


def _node_enc_call(feats, w1, b1, w2, b2, w3, b3, g, bb, pa, pb):
    L = _LATENT
    return pl.pallas_call(
        _node_enc_body,
        grid=(_N // _NBLK,),
        in_specs=[_row_spec(_NBLK, 32), _w_spec(32, L), _w_spec(1, L),
                  _w_spec(L, L), _w_spec(1, L), _w_spec(L, L), _w_spec(1, L),
                  _w_spec(1, L), _w_spec(1, L), _w_spec(L, L), _w_spec(L, L)],
        out_specs=[_row_spec(_NBLK, L), _row_spec(_NBLK, 2 * L)],
        out_shape=[jax.ShapeDtypeStruct((_N, L), _f32),
                   jax.ShapeDtypeStruct((_N, 2 * L), _f32)],
    )(feats, w1, b1, w2, b2, w3, b3, g, bb, pa, pb)


def _edge_step1_body(ps_ref, pr_ref, gs_ref, gr_ref,
                     w1p, w1d, b1e, w2e, b2e, w3e, b3e, ge, bbe,
                     w1, b1, w2, b2, w3, b3, g, bb,
                     enew_ref, eupd_ref):
    d = (ps_ref[...] - pr_ref[...]) * (1.0 / _RADIUS)
    dist = jnp.sqrt(jnp.sum(d * d, axis=-1, keepdims=True) + 1e-12)
    h = _dot(d, w1p[...]) + dist * w1d[...] + b1e[...]
    h = jnp.maximum(h, 0.0)
    h = jnp.maximum(_dot(h, w2e[...]) + b2e[...], 0.0)
    e0 = _ln(_dot(h, w3e[...]) + b3e[...], ge[...], bbe[...])
    h = _dot(e0, w1[...]) + gs_ref[...] + gr_ref[...] + b1[...]
    h = jnp.maximum(h, 0.0)
    h = jnp.maximum(_dot(h, w2[...]) + b2[...], 0.0)
    y = _ln(_dot(h, w3[...]) + b3[...], g[...], bb[...])
    eupd_ref[...] = y
    enew_ref[...] = e0 + y


def _edge_step1_call(pp, gg, encw, stepw):
    L = _LATENT
    off = _E // _EBLK
    return pl.pallas_call(
        _edge_step1_body,
        grid=(_E // _EBLK,),
        in_specs=[_row_spec(_EBLK, L), _row_spec_off(_EBLK, L, off),
                  _row_spec(_EBLK, L), _row_spec_off(_EBLK, L, off),
                  _w_spec(L, L), _w_spec(1, L), _w_spec(1, L),
                  _w_spec(L, L), _w_spec(1, L), _w_spec(L, L), _w_spec(1, L),
                  _w_spec(1, L), _w_spec(1, L),
                  _w_spec(L, L), _w_spec(1, L), _w_spec(L, L), _w_spec(1, L),
                  _w_spec(L, L), _w_spec(1, L), _w_spec(1, L), _w_spec(1, L)],
        out_specs=[_row_spec(_EBLK, L)] * 2,
        out_shape=[jax.ShapeDtypeStruct((_E, L), _f32)] * 2,
    )(pp, pp, gg, gg, *encw, *stepw)


def _edge_step_body(e_ref, gs_ref, gr_ref, w1, b1, w2, b2, w3, b3, g, bb,
                    enew_ref, eupd_ref):
    h = _dot(e_ref[...], w1[...]) + gs_ref[...] + gr_ref[...] + b1[...]
    h = jnp.maximum(h, 0.0)
    h = jnp.maximum(_dot(h, w2[...]) + b2[...], 0.0)
    y = _ln(_dot(h, w3[...]) + b3[...], g[...], bb[...])
    eupd_ref[...] = y
    enew_ref[...] = e_ref[...] + y


def _edge_step_call(e, gg, w1, b1, w2, b2, w3, b3, g, bb):
    L = _LATENT
    ne = e.shape[0]
    off = ne // _EBLK
    return pl.pallas_call(
        _edge_step_body,
        grid=(ne // _EBLK,),
        in_specs=[_row_spec(_EBLK, L), _row_spec(_EBLK, L),
                  _row_spec_off(_EBLK, L, off)] +
                 [_w_spec(L, L), _w_spec(1, L), _w_spec(L, L), _w_spec(1, L),
                  _w_spec(L, L), _w_spec(1, L), _w_spec(1, L), _w_spec(1, L)],
        out_specs=[_row_spec(_EBLK, L)] * 2,
        out_shape=[jax.ShapeDtypeStruct((ne, L), _f32)] * 2,
    )(e, gg, gg, w1, b1, w2, b2, w3, b3, g, bb)


def _node_step_proj_body(x_ref, s0_ref, s1_ref, v1x, v1a, b1,
                         w2, b2, w3, b3, g, bb, pa, pb, xn_ref, t_ref):
    agg = s0_ref[...] + s1_ref[...]
    h = _dot(x_ref[...], v1x[...]) + _dot(agg, v1a[...]) + b1[...]
    h = jnp.maximum(h, 0.0)
    h = jnp.maximum(_dot(h, w2[...]) + b2[...], 0.0)
    y = _ln(_dot(h, w3[...]) + b3[...], g[...], bb[...])
    xn = x_ref[...] + y
    xn_ref[...] = xn
    t_ref[:, :_LATENT] = _dot(xn, pa[...])
    t_ref[:, _LATENT:] = _dot(xn, pb[...])


def _node_step_proj_call(x, ss, v1x, v1a, b1, w2, b2, w3, b3, g, bb, pa, pb):
    L = _LATENT
    return pl.pallas_call(
        _node_step_proj_body,
        grid=(_N // _NBLK,),
        in_specs=[_row_spec(_NBLK, L)] * 3 +
                 [_w_spec(L, L), _w_spec(L, L), _w_spec(1, L), _w_spec(L, L),
                  _w_spec(1, L), _w_spec(L, L), _w_spec(1, L), _w_spec(1, L),
                  _w_spec(1, L), _w_spec(L, L), _w_spec(L, L)],
        out_specs=[_row_spec(_NBLK, L), _row_spec(_NBLK, 2 * L)],
        out_shape=[jax.ShapeDtypeStruct((_N, L), _f32),
                   jax.ShapeDtypeStruct((_N, 2 * L), _f32)],
    )(x, *ss, v1x, v1a, b1, w2, b2, w3, b3, g, bb, pa, pb)


def _node_dec_body(x_ref, s0_ref, s1_ref, v1x, v1a, b1, w2, b2, w3, b3,
                   g, bb, d1, db1, d2, db2, d3, db3, out_ref):
    agg = s0_ref[...] + s1_ref[...]
    h = _dot(x_ref[...], v1x[...]) + _dot(agg, v1a[...]) + b1[...]
    h = jnp.maximum(h, 0.0)
    h = jnp.maximum(_dot(h, w2[...]) + b2[...], 0.0)
    y = _ln(_dot(h, w3[...]) + b3[...], g[...], bb[...])
    xn = x_ref[...] + y
    h = jnp.maximum(_dot(xn, d1[...]) + db1[...], 0.0)
    h = jnp.maximum(_dot(h, d2[...]) + db2[...], 0.0)
    out_ref[...] = _dot(h, d3[...]) + db3[...]


def _node_dec_call(x, ss, v1x, v1a, b1, w2, b2, w3, b3, g, bb,
                   d1, db1, d2, db2, d3, db3):
    L = _LATENT
    return pl.pallas_call(
        _node_dec_body,
        grid=(_N // _NBLK,),
        in_specs=[_row_spec(_NBLK, L)] * 3 +
                 [_w_spec(L, L), _w_spec(L, L), _w_spec(1, L), _w_spec(L, L),
                  _w_spec(1, L), _w_spec(L, L), _w_spec(1, L), _w_spec(1, L),
                  _w_spec(1, L), _w_spec(L, L), _w_spec(1, L), _w_spec(L, L),
                  _w_spec(1, L), _w_spec(L, L), _w_spec(1, L)],
        out_specs=_row_spec(_NBLK, L),
        out_shape=jax.ShapeDtypeStruct((_N, L), _f32),
    )(x, *ss, v1x, v1a, b1, w2, b2, w3, b3, g, bb, d1, db1, d2, db2, d3, db3)


def _r1(v):
    return v.reshape(1, -1)


def kernel(current_positions, edge_index, particle_types, meta_feature, params):
    n = current_positions.shape[0]
    assert n == _N and edge_index.shape[1] == _E

    senders = edge_index[0]
    receivers = edge_index[1]
    idx_pos = jnp.concatenate([senders, receivers])
    idx_lat = jnp.concatenate([2 * senders, 2 * receivers + 1])

    most_recent = current_positions[:, -1]
    flat_vel = (current_positions[:, 1:] - current_positions[:, :-1]).reshape(n, 15)
    npos = (most_recent - 0.5) / 0.5

    nenc = params["node_enc"]["mlp"]
    w1n = nenc[0]["W"]
    meta4 = meta_feature[jnp.array([0, 1, 2, 5])]
    b1n = nenc[0]["b"] + meta4 @ w1n[18:22]
    w1n_eff = jnp.concatenate(
        [w1n[0:18], w1n[22:23], params["embed"] @ w1n[23:39],
         jnp.zeros((4, _LATENT), _f32)], axis=0)
    onehot = (particle_types[:, None] == jnp.arange(9)[None, :]).astype(_f32)
    mcol = jnp.where(particle_types == 1, 10.0, meta_feature[6])
    feats = jnp.concatenate(
        [flat_vel, npos, mcol[:, None], onehot, jnp.zeros((n, 4), _f32)], axis=1)

    gnn = params["gnn"]
    p0 = gnn[0]["edge_mlp"][0]["W"]
    x, xt = _node_enc_call(
        feats, w1n_eff, _r1(b1n), nenc[1]["W"], _r1(nenc[1]["b"]),
        nenc[2]["W"], _r1(nenc[2]["b"]),
        _r1(params["node_enc"]["ln"]["g"]), _r1(params["node_enc"]["ln"]["b"]),
        p0[_LATENT:2 * _LATENT], p0[2 * _LATENT:])

    postable = jnp.concatenate(
        [most_recent, jnp.zeros((n, _LATENT - _DIM), _f32)], axis=1)
    eenc = params["edge_enc"]["mlp"]
    w1e4 = eenc[0]["W"]
    w1p = jnp.concatenate(
        [w1e4[0:3], jnp.zeros((_LATENT - 3, _LATENT), _f32)], axis=0)
    encw = (w1p, w1e4[3:4], _r1(eenc[0]["b"]),
            eenc[1]["W"], _r1(eenc[1]["b"]), eenc[2]["W"], _r1(eenc[2]["b"]),
            _r1(params["edge_enc"]["ln"]["g"]),
            _r1(params["edge_enc"]["ln"]["b"]))
    dec = params["decode"]["mlp"]
    w3d = jnp.zeros((_LATENT, _LATENT), _f32).at[:, :_DIM + 1].set(dec[2]["W"])
    b3d = jnp.zeros((_LATENT,), _f32).at[:_DIM + 1].set(dec[2]["b"])

    zeros_n = jnp.zeros((_NPAD, _LATENT), _f32)
    e = None
    pred = None
    for k in range(len(gnn)):
        blk = gnn[k]
        em = blk["edge_mlp"]
        nm = blk["node_mlp"]
        table = xt.reshape(2 * n, _LATENT)
        estep = (em[0]["W"][:_LATENT], _r1(em[0]["b"]),
                 em[1]["W"], _r1(em[1]["b"]), em[2]["W"], _r1(em[2]["b"]),
                 _r1(blk["edge_ln"]["g"]), _r1(blk["edge_ln"]["b"]))
        if k == 0:
            pp = _sc_gather(postable, idx_pos, _LATENT)
            gg = _sc_gather(table, idx_lat, _LATENT)
            e, e_upd = _edge_step1_call(pp, gg, encw, estep)
        else:
            gg = _sc_gather(table, idx_lat, _LATENT)
            e, e_upd = _edge_step_call(e, gg, *estep)
        s_part = _sc_scatter_add(e_upd, receivers, zeros_n)
        ss = [s_part[:n], s_part[_NPAD:_NPAD + n]]
        nargs = (x, ss, nm[0]["W"][:_LATENT], nm[0]["W"][_LATENT:],
                 _r1(nm[0]["b"]), nm[1]["W"], _r1(nm[1]["b"]),
                 nm[2]["W"], _r1(nm[2]["b"]),
                 _r1(blk["node_ln"]["g"]), _r1(blk["node_ln"]["b"]))
        if k + 1 < len(gnn):
            pn = gnn[k + 1]["edge_mlp"][0]["W"]
            x, xt = _node_step_proj_call(
                *nargs, pn[_LATENT:2 * _LATENT], pn[2 * _LATENT:])
        else:
            pred = _node_dec_call(
                *nargs, dec[0]["W"], _r1(dec[0]["b"]),
                dec[1]["W"], _r1(dec[1]["b"]), w3d, _r1(b3d))
    acc = pred[:, :_DIM] * _ACC_STD + _ACC_MEAN
    recent_vel = most_recent - current_positions[:, -2]
    return most_recent + recent_vel + acc
